# baseline (speedup 1.0000x reference)
import jax
import jax.numpy as jnp
import numpy as np
from jax.experimental import pallas as pl
from jax.experimental.pallas import tpu as pltpu

D = 1024
H = 16
E = 8
EXP = 64
S = 2048
DK = D // H
EF = E * EXP

BS = 1024
QT = 2048
KC = 1024


def _qkv_body(x_ref, wq_ref, bq_ref, wk_ref, bk_ref, wv_ref, bv_ref,
              q_ref, k_ref, v_ref):
    x = x_ref[...]
    q = jnp.dot(x, wq_ref[...], preferred_element_type=jnp.float32) + bq_ref[...]
    k = jnp.dot(x, wk_ref[...], preferred_element_type=jnp.float32) + bk_ref[...]
    q_ref[...] = q.astype(jnp.bfloat16)
    k_ref[...] = k.astype(jnp.bfloat16)
    v_ref[...] = jnp.dot(x, wv_ref[...], preferred_element_type=jnp.float32) + bv_ref[...]


def _attn_body(q_ref, k_ref, v_ref, o_ref):
    outs = []
    for hh in range(2):
        hsl = slice(hh * DK, (hh + 1) * DK)
        q = q_ref[:, hsl]
        m = jnp.full((QT, 1), -jnp.inf, jnp.float32)
        l = jnp.zeros((QT, 1), jnp.float32)
        o = jnp.zeros((QT, DK), jnp.float32)
        for kc in range(S // KC):
            ksl = slice(kc * KC, (kc + 1) * KC)
            kch = k_ref[ksl, hsl]
            vch = v_ref[ksl, hsl]
            s = jnp.dot(q, kch.T, preferred_element_type=jnp.float32) * 0.125
            m_c = jnp.max(s, axis=-1, keepdims=True)
            new_m = jnp.maximum(m, m_c)
            delta = jnp.where(m == new_m, 0.0, m - new_m)
            p = jnp.exp(s - new_m)
            l_c = jnp.sum(p, axis=-1, keepdims=True)
            corr = jnp.exp(delta)
            scale = corr * l
            new_l = scale + l_c
            o = scale * o + jnp.dot(p, vch, preferred_element_type=jnp.float32)
            o = o * (1.0 / new_l)
            m = new_m
            l = new_l
        outs.append(o)
    o_ref[...] = jnp.concatenate(outs, axis=1)


def _tail_body(ao_ref, wo_ref, bo_ref, x_ref, g1_ref, b1_ref, wg_ref,
               bg_ref, we1_ref, be1_ref, we2_ref, be2_ref, g2_ref, b2_ref,
               out_ref):
    ao = jnp.dot(ao_ref[...], wo_ref[...], preferred_element_type=jnp.float32) + bo_ref[...]
    y = x_ref[...] + ao
    m = jnp.mean(y, axis=-1, keepdims=True)
    yc = y - m
    var = jnp.mean(yc * yc, axis=-1, keepdims=True)
    x1 = yc / jnp.sqrt(var + 1e-5) * g1_ref[...] + b1_ref[...]

    logits = jnp.dot(x1, wg_ref[...], preferred_element_type=jnp.float32) + bg_ref[...]
    probs = jax.nn.softmax(logits, axis=-1)
    w1 = jnp.max(probs, axis=-1, keepdims=True)
    i1 = jnp.argmax(probs, axis=-1)[:, None]
    lane = jax.lax.broadcasted_iota(jnp.int32, probs.shape, 1)
    masked = jnp.where(lane == i1, -jnp.inf, probs)
    w2 = jnp.max(masked, axis=-1, keepdims=True)
    i2 = jnp.argmax(masked, axis=-1)[:, None]
    denom = w1 + w2 + 1e-9
    w8 = (jnp.where(lane == i1, w1 / denom, 0.0)
          + jnp.where(lane == i2, w2 / denom, 0.0))

    h = jnp.dot(x1, we1_ref[...], preferred_element_type=jnp.float32) + be1_ref[...]
    h = jnp.maximum(h, 0.0)
    expand = jnp.repeat(jnp.eye(E, dtype=jnp.float32), EXP, axis=1)
    wfull = jnp.dot(w8, expand, preferred_element_type=jnp.float32)
    hw = h * wfull
    moe = (jnp.dot(hw, we2_ref[...], preferred_element_type=jnp.float32)
           + jnp.dot(w8, be2_ref[...], preferred_element_type=jnp.float32))

    y2 = x1 + moe
    m2 = jnp.mean(y2, axis=-1, keepdims=True)
    yc2 = y2 - m2
    var2 = jnp.mean(yc2 * yc2, axis=-1, keepdims=True)
    out_ref[...] = yc2 / jnp.sqrt(var2 + 1e-5) * g2_ref[...] + b2_ref[...]


@jax.jit
def kernel(x, Wq, bq, Wk, bk, Wv, bv, Wo, bo, Wg, bg, We1, be1, We2, be2,
           g1, b1, g2, b2):
    Bz, Sz, Dz = x.shape
    xf = x.reshape(Sz, Dz)
    b2d = lambda b: b.reshape(1, -1)
    We1f = We1.transpose(1, 0, 2).reshape(Dz, EF)
    We2f = We2.reshape(EF, Dz)
    be1f = be1.reshape(1, EF)

    q, k, v = pl.pallas_call(
        _qkv_body,
        grid=(Sz // BS,),
        in_specs=[
            pl.BlockSpec((BS, Dz), lambda i: (i, 0)),
            pl.BlockSpec((Dz, Dz), lambda i: (0, 0)),
            pl.BlockSpec((1, Dz), lambda i: (0, 0)),
            pl.BlockSpec((Dz, Dz), lambda i: (0, 0)),
            pl.BlockSpec((1, Dz), lambda i: (0, 0)),
            pl.BlockSpec((Dz, Dz), lambda i: (0, 0)),
            pl.BlockSpec((1, Dz), lambda i: (0, 0)),
        ],
        out_specs=[
            pl.BlockSpec((BS, Dz), lambda i: (i, 0)),
            pl.BlockSpec((BS, Dz), lambda i: (i, 0)),
            pl.BlockSpec((BS, Dz), lambda i: (i, 0)),
        ],
        out_shape=[jax.ShapeDtypeStruct((Sz, Dz), jnp.bfloat16),
                   jax.ShapeDtypeStruct((Sz, Dz), jnp.bfloat16),
                   jax.ShapeDtypeStruct((Sz, Dz), jnp.float32)],
    )(xf, Wq, b2d(bq), Wk, b2d(bk), Wv, b2d(bv))

    ao = pl.pallas_call(
        _attn_body,
        grid=(H // 2, Sz // QT),
        in_specs=[
            pl.BlockSpec((QT, 2 * DK), lambda h, i: (i, h)),
            pl.BlockSpec((Sz, 2 * DK), lambda h, i: (0, h)),
            pl.BlockSpec((Sz, 2 * DK), lambda h, i: (0, h)),
        ],
        out_specs=pl.BlockSpec((QT, 2 * DK), lambda h, i: (i, h)),
        out_shape=jax.ShapeDtypeStruct((Sz, Dz), jnp.float32),
    )(q, k, v)

    out = pl.pallas_call(
        _tail_body,
        grid=(Sz // BS,),
        in_specs=[
            pl.BlockSpec((BS, Dz), lambda i: (i, 0)),
            pl.BlockSpec((Dz, Dz), lambda i: (0, 0)),
            pl.BlockSpec((1, Dz), lambda i: (0, 0)),
            pl.BlockSpec((BS, Dz), lambda i: (i, 0)),
            pl.BlockSpec((1, Dz), lambda i: (0, 0)),
            pl.BlockSpec((1, Dz), lambda i: (0, 0)),
            pl.BlockSpec((Dz, E), lambda i: (0, 0)),
            pl.BlockSpec((1, E), lambda i: (0, 0)),
            pl.BlockSpec((Dz, EF), lambda i: (0, 0)),
            pl.BlockSpec((1, EF), lambda i: (0, 0)),
            pl.BlockSpec((EF, Dz), lambda i: (0, 0)),
            pl.BlockSpec((E, Dz), lambda i: (0, 0)),
            pl.BlockSpec((1, Dz), lambda i: (0, 0)),
            pl.BlockSpec((1, Dz), lambda i: (0, 0)),
        ],
        out_specs=pl.BlockSpec((BS, Dz), lambda i: (i, 0)),
        out_shape=jax.ShapeDtypeStruct((Sz, Dz), jnp.float32),
    )(ao, Wo, b2d(bo), xf, b2d(g1), b2d(b1), Wg, b2d(bg),
      We1f, be1f, We2f, be2, b2d(g2), b2d(b2))

    return out.reshape(Bz, Sz, Dz)


# speedup vs baseline: 1.0228x; 1.0228x over previous
import jax
import jax.numpy as jnp
import numpy as np
from jax.experimental import pallas as pl
from jax.experimental.pallas import tpu as pltpu

D = 1024
H = 16
E = 8
EXP = 64
S = 2048
DK = D // H
EF = E * EXP

BS = 512
QT = 2048
KC = 1024


def _qkv_body(x_ref, wq_ref, bq_ref, wk_ref, bk_ref, wv_ref, bv_ref,
              q_ref, k_ref, v_ref):
    x = x_ref[...]
    q = jnp.dot(x, wq_ref[...], preferred_element_type=jnp.float32) + bq_ref[...]
    k = jnp.dot(x, wk_ref[...], preferred_element_type=jnp.float32) + bk_ref[...]
    q_ref[...] = q.astype(jnp.bfloat16)
    k_ref[...] = k.astype(jnp.bfloat16)
    v = jnp.dot(x, wv_ref[...], preferred_element_type=jnp.float32) + bv_ref[...]
    v_ref[...] = v.astype(jnp.bfloat16)


def _attn_body(q_ref, k_ref, v_ref, o_ref):
    outs = []
    for hh in range(2):
        hsl = slice(hh * DK, (hh + 1) * DK)
        q = q_ref[:, hsl]
        m = jnp.full((QT, 1), -jnp.inf, jnp.float32)
        l = jnp.zeros((QT, 1), jnp.float32)
        o = jnp.zeros((QT, DK), jnp.float32)
        for kc in range(S // KC):
            ksl = slice(kc * KC, (kc + 1) * KC)
            kch = k_ref[ksl, hsl]
            vch = v_ref[ksl, hsl]
            s = jnp.dot(q, kch.T, preferred_element_type=jnp.float32) * 0.125
            m_c = jnp.max(s, axis=-1, keepdims=True)
            new_m = jnp.maximum(m, m_c)
            delta = jnp.where(m == new_m, 0.0, m - new_m)
            p = jnp.exp(s - new_m)
            l_c = jnp.sum(p, axis=-1, keepdims=True)
            corr = jnp.exp(delta)
            scale = corr * l
            new_l = scale + l_c
            o = scale * o + jnp.dot(p.astype(jnp.bfloat16), vch,
                                    preferred_element_type=jnp.float32)
            o = o * (1.0 / new_l)
            m = new_m
            l = new_l
        outs.append(o)
    o_ref[...] = jnp.concatenate(outs, axis=1)


def _tail_body(ao_ref, wo_ref, bo_ref, x_ref, g1_ref, b1_ref, wg_ref,
               bg_ref, we1_ref, be1_ref, we2_ref, be2_ref, g2_ref, b2_ref,
               out_ref):
    ao = jnp.dot(ao_ref[...], wo_ref[...], preferred_element_type=jnp.float32) + bo_ref[...]
    y = x_ref[...] + ao
    m = jnp.mean(y, axis=-1, keepdims=True)
    yc = y - m
    var = jnp.mean(yc * yc, axis=-1, keepdims=True)
    x1 = yc / jnp.sqrt(var + 1e-5) * g1_ref[...] + b1_ref[...]

    logits = jnp.dot(x1, wg_ref[...], preferred_element_type=jnp.float32) + bg_ref[...]
    probs = jax.nn.softmax(logits, axis=-1)
    w1 = jnp.max(probs, axis=-1, keepdims=True)
    i1 = jnp.argmax(probs, axis=-1)[:, None]
    lane = jax.lax.broadcasted_iota(jnp.int32, probs.shape, 1)
    masked = jnp.where(lane == i1, -jnp.inf, probs)
    w2 = jnp.max(masked, axis=-1, keepdims=True)
    i2 = jnp.argmax(masked, axis=-1)[:, None]
    denom = w1 + w2 + 1e-9
    w8 = (jnp.where(lane == i1, w1 / denom, 0.0)
          + jnp.where(lane == i2, w2 / denom, 0.0))

    h = jnp.dot(x1, we1_ref[...], preferred_element_type=jnp.float32) + be1_ref[...]
    h = jnp.maximum(h, 0.0)
    expand = jnp.repeat(jnp.eye(E, dtype=jnp.float32), EXP, axis=1)
    wfull = jnp.dot(w8, expand, preferred_element_type=jnp.float32)
    hw = h * wfull
    moe = (jnp.dot(hw, we2_ref[...], preferred_element_type=jnp.float32)
           + jnp.dot(w8, be2_ref[...], preferred_element_type=jnp.float32))

    y2 = x1 + moe
    m2 = jnp.mean(y2, axis=-1, keepdims=True)
    yc2 = y2 - m2
    var2 = jnp.mean(yc2 * yc2, axis=-1, keepdims=True)
    out_ref[...] = yc2 / jnp.sqrt(var2 + 1e-5) * g2_ref[...] + b2_ref[...]


@jax.jit
def kernel(x, Wq, bq, Wk, bk, Wv, bv, Wo, bo, Wg, bg, We1, be1, We2, be2,
           g1, b1, g2, b2):
    Bz, Sz, Dz = x.shape
    xf = x.reshape(Sz, Dz)
    b2d = lambda b: b.reshape(1, -1)
    We1f = We1.transpose(1, 0, 2).reshape(Dz, EF)
    We2f = We2.reshape(EF, Dz)
    be1f = be1.reshape(1, EF)

    q, k, v = pl.pallas_call(
        _qkv_body,
        grid=(Sz // BS,),
        in_specs=[
            pl.BlockSpec((BS, Dz), lambda i: (i, 0)),
            pl.BlockSpec((Dz, Dz), lambda i: (0, 0)),
            pl.BlockSpec((1, Dz), lambda i: (0, 0)),
            pl.BlockSpec((Dz, Dz), lambda i: (0, 0)),
            pl.BlockSpec((1, Dz), lambda i: (0, 0)),
            pl.BlockSpec((Dz, Dz), lambda i: (0, 0)),
            pl.BlockSpec((1, Dz), lambda i: (0, 0)),
        ],
        out_specs=[
            pl.BlockSpec((BS, Dz), lambda i: (i, 0)),
            pl.BlockSpec((BS, Dz), lambda i: (i, 0)),
            pl.BlockSpec((BS, Dz), lambda i: (i, 0)),
        ],
        out_shape=[jax.ShapeDtypeStruct((Sz, Dz), jnp.bfloat16)] * 3,
    )(xf, Wq, b2d(bq), Wk, b2d(bk), Wv, b2d(bv))

    ao = pl.pallas_call(
        _attn_body,
        grid=(H // 2, Sz // QT),
        in_specs=[
            pl.BlockSpec((QT, 2 * DK), lambda h, i: (i, h)),
            pl.BlockSpec((Sz, 2 * DK), lambda h, i: (0, h)),
            pl.BlockSpec((Sz, 2 * DK), lambda h, i: (0, h)),
        ],
        out_specs=pl.BlockSpec((QT, 2 * DK), lambda h, i: (i, h)),
        out_shape=jax.ShapeDtypeStruct((Sz, Dz), jnp.float32),
    )(q, k, v)

    out = pl.pallas_call(
        _tail_body,
        grid=(Sz // BS,),
        in_specs=[
            pl.BlockSpec((BS, Dz), lambda i: (i, 0)),
            pl.BlockSpec((Dz, Dz), lambda i: (0, 0)),
            pl.BlockSpec((1, Dz), lambda i: (0, 0)),
            pl.BlockSpec((BS, Dz), lambda i: (i, 0)),
            pl.BlockSpec((1, Dz), lambda i: (0, 0)),
            pl.BlockSpec((1, Dz), lambda i: (0, 0)),
            pl.BlockSpec((Dz, E), lambda i: (0, 0)),
            pl.BlockSpec((1, E), lambda i: (0, 0)),
            pl.BlockSpec((Dz, EF), lambda i: (0, 0)),
            pl.BlockSpec((1, EF), lambda i: (0, 0)),
            pl.BlockSpec((EF, Dz), lambda i: (0, 0)),
            pl.BlockSpec((E, Dz), lambda i: (0, 0)),
            pl.BlockSpec((1, Dz), lambda i: (0, 0)),
            pl.BlockSpec((1, Dz), lambda i: (0, 0)),
        ],
        out_specs=pl.BlockSpec((BS, Dz), lambda i: (i, 0)),
        out_shape=jax.ShapeDtypeStruct((Sz, Dz), jnp.float32),
    )(ao, Wo, b2d(bo), xf, b2d(g1), b2d(b1), Wg, b2d(bg),
      We1f, be1f, We2f, be2, b2d(g2), b2d(b2))

    return out.reshape(Bz, Sz, Dz)
